# baseline (speedup 1.0000x reference)
import jax
import jax.numpy as jnp
from jax.experimental import pallas as pl

_NODE_IN, _EDGE_IN, _HID = 128, 16, 500
_HP = 512
_B_RXN, _C, _N_PER, _E_PER = 50, 20, 50, 100
_V1, _E1 = _B_RXN * _N_PER, _B_RXN * _E_PER
_B = _B_RXN * _C
_V2, _E2 = _B * _N_PER, _B * _E_PER
_N_LAYERS = 3
_NP_, _EP_ = _C * _N_PER, _C * _E_PER
_S_P = 2

_F32 = jnp.float32


def _pad2(w, rows, cols):
    return jnp.pad(w, ((0, rows - w.shape[0]), (0, cols - w.shape[1])))


def _pad_bias(b):
    return jnp.pad(b, (0, _HP - b.shape[0])).reshape(1, _HP)


def _onehots(src_row, dst_row, n_blk, n_sub, e_sub):
    iota_n = jax.lax.broadcasted_iota(jnp.int32, (n_sub, e_sub), 0)
    ohg, ohd = [], []
    for s in range(n_blk // n_sub):
        src_s = src_row[:, s * e_sub:(s + 1) * e_sub] - (s * n_sub)
        dst_s = dst_row[:, s * e_sub:(s + 1) * e_sub] - (s * n_sub)
        ohg.append((jnp.broadcast_to(src_s, (n_sub, e_sub)) == iota_n)
                   .astype(_F32))
        ohd.append((jnp.broadcast_to(dst_s, (n_sub, e_sub)) == iota_n)
                   .astype(_F32))
    return ohg, ohd


def _msum(hw, ew, ohg, ohd, n_sub, e_sub):
    msums = []
    for s in range(len(ohg)):
        hw_s = hw[s * n_sub:(s + 1) * n_sub, :]
        hsrc = jax.lax.dot_general(ohg[s], hw_s, (((0,), (0,)), ((), ())),
                                   preferred_element_type=_F32)
        msg = jnp.maximum(hsrc + ew[s * e_sub:(s + 1) * e_sub, :], 0.0)
        msums.append(jnp.dot(ohd[s], msg, preferred_element_type=_F32))
    return jnp.concatenate(msums, axis=0) if len(msums) > 1 else msums[0]


def _body(rnf, pnf, ref_, pef, rsrcr, rdstr, psrcr, pdstr, cs, *args):
    out = args[-1]
    pW, pb = args[0], args[1]
    lw = args[2:2 + 6 * _N_LAYERS]
    dmW1, dmW2, dmb, dnW1, dnW2, dnb, pW1, pb1, pW2, pb2 = \
        args[2 + 6 * _N_LAYERS:-1]
    r = pl.program_id(0)
    rs_row = rsrcr[0] - r * _N_PER
    rd_row = rdstr[0] - r * _N_PER
    ps_row = psrcr[0] - r * _NP_
    pd_row = pdstr[0] - r * _NP_
    ohg_r, ohd_r = _onehots(rs_row, rd_row, _N_PER, _N_PER, _E_PER)
    n_sub, e_sub = _S_P * _N_PER, _S_P * _E_PER
    ohg_p, ohd_p = _onehots(ps_row, pd_row, _NP_, n_sub, e_sub)

    h_r = jnp.maximum(jnp.dot(rnf[0], pW[...], preferred_element_type=_F32)
                      + pb[...], 0.0)
    h_p = jnp.maximum(jnp.dot(pnf[0], pW[...], preferred_element_type=_F32)
                      + pb[...], 0.0)
    er, ep = ref_[0], pef[0]
    for i in range(_N_LAYERS):
        mW1, mW2, mb, nW1, nW2, nb = lw[6 * i:6 * i + 6]
        ewr = jnp.dot(er, mW2[...], preferred_element_type=_F32) + mb[...]
        ewp = jnp.dot(ep, mW2[...], preferred_element_type=_F32) + mb[...]
        msr = _msum(jnp.dot(h_r, mW1[...], preferred_element_type=_F32),
                    ewr, ohg_r, ohd_r, _N_PER, _E_PER)
        msp = _msum(jnp.dot(h_p, mW1[...], preferred_element_type=_F32),
                    ewp, ohg_p, ohd_p, n_sub, e_sub)
        h_r = jnp.maximum(jnp.dot(h_r, nW1[...], preferred_element_type=_F32)
                          + jnp.dot(msr, nW2[...],
                                    preferred_element_type=_F32)
                          + nb[...], 0.0)
        h_p = jnp.maximum(jnp.dot(h_p, nW1[...], preferred_element_type=_F32)
                          + jnp.dot(msp, nW2[...],
                                    preferred_element_type=_F32)
                          + nb[...], 0.0)
    n_i = jax.lax.broadcasted_iota(jnp.int32, (_NP_, _N_PER), 0)
    i_i = jax.lax.broadcasted_iota(jnp.int32, (_NP_, _N_PER), 1)
    rep_oh = (n_i % _N_PER == i_i).astype(_F32)
    diff = h_p - jnp.dot(rep_oh, h_r, preferred_element_type=_F32)
    ew_d = jnp.dot(ep, dmW2[...], preferred_element_type=_F32) + dmb[...]
    msd = _msum(jnp.dot(diff, dmW1[...], preferred_element_type=_F32),
                ew_d, ohg_p, ohd_p, n_sub, e_sub)
    h2 = jnp.maximum(jnp.dot(diff, dnW1[...], preferred_element_type=_F32)
                     + jnp.dot(msd, dnW2[...], preferred_element_type=_F32)
                     + dnb[...], 0.0)
    g_i = jax.lax.broadcasted_iota(jnp.int32, (_C, _NP_), 0)
    n_i2 = jax.lax.broadcasted_iota(jnp.int32, (_C, _NP_), 1)
    sum_oh = (n_i2 // _N_PER == g_i).astype(_F32)
    readout = jnp.dot(sum_oh, h2, preferred_element_type=_F32)
    hidden = jnp.maximum(jnp.dot(readout, pW1[...], preferred_element_type=_F32)
                         + pb1[...], 0.0)
    out[0] = (jnp.dot(hidden, pW2[...], preferred_element_type=_F32)
              + pb2[...] + cs[0])


def _full_spec(shape):
    nd = len(shape)
    return pl.BlockSpec(shape, lambda i: (0,) * nd)


def kernel(reactant_node_feats, reactant_edge_feats, product_node_feats,
           product_edge_feats, candidate_scores, reactant_edge_index,
           product_edge_index, params):
    p = params
    rnf3 = reactant_node_feats.reshape(_B_RXN, _N_PER, _NODE_IN)
    pnf3 = product_node_feats.reshape(_B_RXN, _NP_, _NODE_IN)
    ref3 = reactant_edge_feats.reshape(_B_RXN, _E_PER, _EDGE_IN)
    pef3 = product_edge_feats.reshape(_B_RXN, _EP_, _EDGE_IN)
    rsrc = reactant_edge_index[0].reshape(_B_RXN, 1, _E_PER)
    rdst = reactant_edge_index[1].reshape(_B_RXN, 1, _E_PER)
    psrc = product_edge_index[0].reshape(_B_RXN, 1, _EP_)
    pdst = product_edge_index[1].reshape(_B_RXN, 1, _EP_)
    cs3 = jnp.pad(candidate_scores, ((0, 0), (0, 127))).reshape(_B_RXN, _C, 128)

    pW = _pad2(p['proj_W'], _NODE_IN, _HP)
    pb = _pad_bias(p['proj_b'])
    ws = [pW, pb]
    for i in range(_N_LAYERS):
        mW = p['msg_W_%d' % i]
        nW = p['node_W_%d' % i]
        ws += [_pad2(mW[:_HID], _HP, _HP), _pad2(mW[_HID:], _EDGE_IN, _HP),
               _pad_bias(p['msg_b_%d' % i]), _pad2(nW[:_HID], _HP, _HP),
               _pad2(nW[_HID:], _HP, _HP), _pad_bias(p['node_b_%d' % i])]
    dmW = p['dmsg_W']
    dnW = p['dnode_W']
    ws += [_pad2(dmW[:_HID], _HP, _HP), _pad2(dmW[_HID:], _EDGE_IN, _HP),
           _pad_bias(p['dmsg_b']), _pad2(dnW[:_HID], _HP, _HP),
           _pad2(dnW[_HID:], _HP, _HP), _pad_bias(p['dnode_b']),
           _pad2(p['pW1'], _HP, _HP), _pad_bias(p['pb1']),
           _pad2(p['pW2'], _HP, 128),
           jnp.pad(p['pb2'], (0, 127)).reshape(1, 128)]

    specs = (
        [pl.BlockSpec((1, _N_PER, _NODE_IN), lambda i: (i, 0, 0)),
         pl.BlockSpec((1, _NP_, _NODE_IN), lambda i: (i, 0, 0)),
         pl.BlockSpec((1, _E_PER, _EDGE_IN), lambda i: (i, 0, 0)),
         pl.BlockSpec((1, _EP_, _EDGE_IN), lambda i: (i, 0, 0)),
         pl.BlockSpec((1, 1, _E_PER), lambda i: (i, 0, 0)),
         pl.BlockSpec((1, 1, _E_PER), lambda i: (i, 0, 0)),
         pl.BlockSpec((1, 1, _EP_), lambda i: (i, 0, 0)),
         pl.BlockSpec((1, 1, _EP_), lambda i: (i, 0, 0)),
         pl.BlockSpec((1, _C, 128), lambda i: (i, 0, 0))]
        + [_full_spec(w.shape) for w in ws])

    scores = pl.pallas_call(
        _body,
        grid=(_B_RXN,),
        in_specs=specs,
        out_specs=pl.BlockSpec((1, _C, 128), lambda i: (i, 0, 0)),
        out_shape=jax.ShapeDtypeStruct((_B_RXN, _C, 128), _F32),
    )(rnf3, pnf3, ref3, pef3, rsrc, rdst, psrc, pdst, cs3, *ws)

    return scores.reshape(_B, 128)[:, :1]
